# baseline (speedup 1.0000x reference)
import jax
import jax.numpy as jnp
from jax.experimental import pallas as pl
from jax.experimental.pallas import tpu as pltpu

NUM_NOISE = 16

BM = 256


def _sim_kernel(x_ref, nz_ref, pos_ref, neg_ref, sim_ref, nsim_ref):
    dims = (((1,), (1,)), ((), ()))
    a = x_ref[0]
    npos = pos_ref.shape[0]
    sim_ref[:, :npos] = jax.lax.dot_general(
        a, pos_ref[...], dims, preferred_element_type=jnp.float32
    )
    sim_ref[:, npos:] = jax.lax.dot_general(
        a, neg_ref[...], dims, preferred_element_type=jnp.float32
    )

    @pl.when(pl.program_id(0) == 0)
    def _():
        nz = nz_ref[...].reshape(-1, nz_ref.shape[-1])
        nsim_ref[...] = jax.lax.dot_general(
            nz, pos_ref[...], dims, preferred_element_type=jnp.float32
        )


def kernel(x, visible, vis_mask, memory_pos, memory_neg):
    b, k, c = x.shape
    num_pos = k - NUM_NOISE
    m = b * num_pos
    n = memory_pos.shape[0] + memory_neg.shape[0]
    blocks_per_batch = num_pos // BM

    sim, nsim = pl.pallas_call(
        _sim_kernel,
        grid=(m // BM,),
        in_specs=[
            pl.BlockSpec(
                (1, BM, c),
                lambda i: (i // blocks_per_batch, i % blocks_per_batch, 0),
            ),
            pl.BlockSpec((b, NUM_NOISE, c), lambda i: (0, num_pos // NUM_NOISE, 0)),
            pl.BlockSpec(memory_pos.shape, lambda i: (0, 0)),
            pl.BlockSpec(memory_neg.shape, lambda i: (0, 0)),
        ],
        out_specs=[
            pl.BlockSpec((BM, n), lambda i: (i, 0)),
            pl.BlockSpec((b * NUM_NOISE, memory_pos.shape[0]), lambda i: (0, 0)),
        ],
        out_shape=[
            jax.ShapeDtypeStruct((m, n), jnp.float32),
            jax.ShapeDtypeStruct((b * NUM_NOISE, memory_pos.shape[0]), jnp.float32),
        ],
        compiler_params=pltpu.CompilerParams(dimension_semantics=("arbitrary",)),
    )(x, x, memory_pos, memory_neg)

    return sim, nsim.reshape(b, NUM_NOISE, -1)


# speedup vs baseline: 1.0069x; 1.0069x over previous
import jax
import jax.numpy as jnp
from jax.experimental import pallas as pl
from jax.experimental.pallas import tpu as pltpu

NUM_NOISE = 16

BM = 512


def _sim_kernel(x_ref, nz_ref, pos_ref, neg_ref, sim_ref, nsim_ref):
    dims = (((1,), (1,)), ((), ()))
    a = x_ref[0]
    npos = pos_ref.shape[0]
    sim_ref[:, :npos] = jax.lax.dot_general(
        a, pos_ref[...], dims, preferred_element_type=jnp.float32
    )
    sim_ref[:, npos:] = jax.lax.dot_general(
        a, neg_ref[...], dims, preferred_element_type=jnp.float32
    )

    @pl.when(pl.program_id(0) == 0)
    def _():
        nz = nz_ref[...].reshape(-1, nz_ref.shape[-1])
        nsim_ref[...] = jax.lax.dot_general(
            nz, pos_ref[...], dims, preferred_element_type=jnp.float32
        )


def kernel(x, visible, vis_mask, memory_pos, memory_neg):
    b, k, c = x.shape
    num_pos = k - NUM_NOISE
    m = b * num_pos
    n = memory_pos.shape[0] + memory_neg.shape[0]
    blocks_per_batch = num_pos // BM

    sim, nsim = pl.pallas_call(
        _sim_kernel,
        grid=(m // BM,),
        in_specs=[
            pl.BlockSpec(
                (1, BM, c),
                lambda i: (i // blocks_per_batch, i % blocks_per_batch, 0),
            ),
            pl.BlockSpec((b, NUM_NOISE, c), lambda i: (0, num_pos // NUM_NOISE, 0)),
            pl.BlockSpec(memory_pos.shape, lambda i: (0, 0)),
            pl.BlockSpec(memory_neg.shape, lambda i: (0, 0)),
        ],
        out_specs=[
            pl.BlockSpec((BM, n), lambda i: (i, 0)),
            pl.BlockSpec((b * NUM_NOISE, memory_pos.shape[0]), lambda i: (0, 0)),
        ],
        out_shape=[
            jax.ShapeDtypeStruct((m, n), jnp.float32),
            jax.ShapeDtypeStruct((b * NUM_NOISE, memory_pos.shape[0]), jnp.float32),
        ],
        compiler_params=pltpu.CompilerParams(dimension_semantics=("arbitrary",)),
    )(x, x, memory_pos, memory_neg)

    return sim, nsim.reshape(b, NUM_NOISE, -1)
